# baseline (speedup 1.0000x reference)
import jax
import jax.numpy as jnp
from jax import lax
from jax.experimental import pallas as pl
from jax.experimental.pallas import tpu as pltpu
from jax.experimental.pallas import tpu_sc as plsc

NC = 2
NS = 16
L = 16
NW = NC * NS

BATCH = 16384
DIM = 32
ROWS_PER_LINE = 4
B_PER_W = BATCH // NW
CHUNK = 128
N_CHUNKS = B_PER_W // CHUNK
GROUPS = CHUNK // L


def _body(uid_hbm, cid_hbm, umat_hbm, imat_hbm, out_hbm,
          uids_v, cids_v, ulidx_v, clidx_v, ubuf_v, ibuf_v, out_v, sem):
  wid = lax.axis_index("s") * NC + lax.axis_index("c")
  base = wid * B_PER_W

  for j in range(N_CHUNKS):
    pltpu.sync_copy(uid_hbm.at[pl.ds(base + j * CHUNK, CHUNK)], uids_v.at[j])
    pltpu.sync_copy(cid_hbm.at[pl.ds(base + j * CHUNK, CHUNK)], cids_v.at[j])
  for j in range(N_CHUNKS):
    for k in range(CHUNK // L):
      s = pl.ds(k * L, L)
      ulidx_v[j, s] = lax.shift_right_logical(uids_v[j, s], 2)
      clidx_v[j, s] = lax.shift_right_logical(cids_v[j, s], 2)

  def start(j):
    buf = j % 2
    return (
        pltpu.async_copy(umat_hbm.at[ulidx_v.at[j]], ubuf_v.at[buf], sem),
        pltpu.async_copy(imat_hbm.at[clidx_v.at[j]], ibuf_v.at[buf], sem),
    )

  lanes = lax.iota(jnp.int32, L)
  three = jnp.full((L,), 3, jnp.int32)

  def compute(j):
    buf = j % 2
    for g in range(GROUPS):
      s = pl.ds(g * L, L)
      ucol0 = lax.shift_left(uids_v[j, s] & three, 5)
      ccol0 = lax.shift_left(cids_v[j, s] & three, 5)
      rows = jnp.full((L,), g * L, jnp.int32) + lanes
      acc = jnp.zeros((L,), jnp.float32)
      for d in range(DIM):
        dv = jnp.full((L,), d, jnp.int32)
        u = plsc.load_gather(ubuf_v.at[buf], [rows, ucol0 + dv])
        v = plsc.load_gather(ibuf_v.at[buf], [rows, ccol0 + dv])
        acc = acc + u * v
      e = jnp.exp(-jnp.abs(acc))
      sig = jnp.where(acc >= 0, 1.0 / (1.0 + e), e / (1.0 + e))
      out_v[pl.ds(j * CHUNK + g * L, L)] = sig

  pending = start(0)
  for j in range(N_CHUNKS):
    for c in pending:
      c.wait()
    if j + 1 < N_CHUNKS:
      pending = start(j + 1)
    compute(j)

  pltpu.sync_copy(out_v, out_hbm.at[pl.ds(base, B_PER_W)])


@jax.jit
def kernel(user_ids, content_ids, user_matrix, item_matrix):
  uid = user_ids.astype(jnp.int32)
  cid = content_ids.astype(jnp.int32)
  umat = user_matrix.reshape(-1, ROWS_PER_LINE * DIM)
  imat = item_matrix.reshape(-1, ROWS_PER_LINE * DIM)

  mesh = plsc.VectorSubcoreMesh(
      core_axis_name="c", subcore_axis_name="s", num_cores=NC,
      num_subcores=NS)

  run = pl.kernel(
      _body,
      out_type=jax.ShapeDtypeStruct((BATCH,), jnp.float32),
      mesh=mesh,
      compiler_params=pltpu.CompilerParams(needs_layout_passes=False),
      scratch_types=[
          pltpu.VMEM((N_CHUNKS, CHUNK), jnp.int32),
          pltpu.VMEM((N_CHUNKS, CHUNK), jnp.int32),
          pltpu.VMEM((N_CHUNKS, CHUNK), jnp.int32),
          pltpu.VMEM((N_CHUNKS, CHUNK), jnp.int32),
          pltpu.VMEM((2, CHUNK, ROWS_PER_LINE * DIM), jnp.float32),
          pltpu.VMEM((2, CHUNK, ROWS_PER_LINE * DIM), jnp.float32),
          pltpu.VMEM((B_PER_W,), jnp.float32),
          pltpu.SemaphoreType.DMA,
      ],
  )
  return run(uid, cid, umat, imat)
